# baseline (speedup 1.0000x reference)
import jax
import jax.numpy as jnp
from jax.experimental import pallas as pl
from jax.experimental.pallas import tpu as pltpu

M = 4096
N = 4096
BM = 128
S = 2
NB = M // BM // S


def _body(*refs):
    a_refs = refs[:S]
    xr_ref = refs[S]
    b_refs = refs[S + 1:2 * S + 1]
    iy_refs = refs[2 * S + 1:3 * S + 1]
    l_ref, u_ref, il_ref, iu_ref, o_ref, acc_ref = refs[3 * S + 1:]
    i = pl.program_id(0)

    @pl.when(i == 0)
    def _init():
        xv = xr_ref[...]
        var = (jnp.maximum(l_ref[...] - xv, 0.0) * il_ref[...]
               + jnp.maximum(xv - u_ref[...], 0.0) * iu_ref[...])
        acc_ref[0:1, :] = jnp.full((1, 128), jnp.max(jnp.abs(var)), jnp.float32)
        acc_ref[1:2, :] = jnp.zeros((1, 128), jnp.float32)
        acc_ref[2:3, :] = jnp.zeros((1, 128), jnp.float32)

    for ar, br, iyr in zip(a_refs, b_refs, iy_refs):
        ax = jnp.sum(ar[...] * xr_ref[...], axis=1)
        bv = br[...]
        cons = bv - ax
        cons = cons + jnp.maximum(-cons, 0.0) * iyr[...]
        acc_ref[0:1, :] = jnp.maximum(
            acc_ref[0:1, :], jnp.full((1, 128), jnp.max(jnp.abs(cons)), jnp.float32))
        acc_ref[1:2, :] = jnp.maximum(
            acc_ref[1:2, :], jnp.full((1, 128), jnp.max(jnp.abs(ax)), jnp.float32))
        acc_ref[2:3, :] = jnp.maximum(
            acc_ref[2:3, :], jnp.full((1, 128), jnp.max(jnp.abs(bv)), jnp.float32))

    @pl.when(i == NB - 1)
    def _finish():
        stk = jnp.max(acc_ref[0:1, :])
        axm = jnp.max(acc_ref[1:2, :])
        bmx = jnp.max(acc_ref[2:3, :])
        o_ref[...] = jnp.reshape(stk / (1.0 + jnp.maximum(axm, bmx)), (1, 1))


_in_specs = (
    [pl.BlockSpec((BM, N), lambda i, s=s: (s * NB + i, 0)) for s in range(S)]
    + [pl.BlockSpec((1, N), lambda i: (0, 0))]
    + [pl.BlockSpec((BM,), lambda i, s=s: (s * NB + i,)) for s in range(S)]
    + [pl.BlockSpec((BM,), lambda i, s=s: (s * NB + i,)) for s in range(S)]
    + [pl.BlockSpec((1, N), lambda i: (0, 0)) for _ in range(4)]
)

_fused = pl.pallas_call(
    _body,
    grid=(NB,),
    in_specs=_in_specs,
    out_specs=pl.BlockSpec((1, 1), lambda i: (0, 0)),
    out_shape=jax.ShapeDtypeStruct((1, 1), jnp.float32),
    scratch_shapes=[pltpu.VMEM((3, 128), jnp.float32)],
)


def kernel(A, b, c, x, Iy, il, iu, l, u):
    del c
    iy = Iy.reshape(M)
    args = ([A] * S + [x.reshape(1, N)] + [b] * S + [iy] * S
            + [l.reshape(1, N), u.reshape(1, N),
               il.reshape(1, N), iu.reshape(1, N)])
    out = _fused(*args)
    return out[0, 0]


# speedup vs baseline: 1.1738x; 1.1738x over previous
import jax
import jax.numpy as jnp
from jax.experimental import pallas as pl
from jax.experimental.pallas import tpu as pltpu

M = 4096
N = 4096
BM = 256
S = 2
NB = M // BM // S


def _body(*refs):
    a_refs = refs[:S]
    xr_ref = refs[S]
    b_refs = refs[S + 1:2 * S + 1]
    iy_refs = refs[2 * S + 1:3 * S + 1]
    l_ref, u_ref, il_ref, iu_ref, o_ref, acc_ref = refs[3 * S + 1:]
    i = pl.program_id(0)

    @pl.when(i == 0)
    def _init():
        xv = xr_ref[...]
        var = (jnp.maximum(l_ref[...] - xv, 0.0) * il_ref[...]
               + jnp.maximum(xv - u_ref[...], 0.0) * iu_ref[...])
        acc_ref[0:1, :] = jnp.full((1, 128), jnp.max(jnp.abs(var)), jnp.float32)
        acc_ref[1:2, :] = jnp.zeros((1, 128), jnp.float32)
        acc_ref[2:3, :] = jnp.zeros((1, 128), jnp.float32)

    for ar, br, iyr in zip(a_refs, b_refs, iy_refs):
        ax = jnp.sum(ar[...] * xr_ref[...], axis=1)
        bv = br[...]
        cons = bv - ax
        cons = cons + jnp.maximum(-cons, 0.0) * iyr[...]
        acc_ref[0:1, :] = jnp.maximum(
            acc_ref[0:1, :], jnp.full((1, 128), jnp.max(jnp.abs(cons)), jnp.float32))
        acc_ref[1:2, :] = jnp.maximum(
            acc_ref[1:2, :], jnp.full((1, 128), jnp.max(jnp.abs(ax)), jnp.float32))
        acc_ref[2:3, :] = jnp.maximum(
            acc_ref[2:3, :], jnp.full((1, 128), jnp.max(jnp.abs(bv)), jnp.float32))

    @pl.when(i == NB - 1)
    def _finish():
        stk = jnp.max(acc_ref[0:1, :])
        axm = jnp.max(acc_ref[1:2, :])
        bmx = jnp.max(acc_ref[2:3, :])
        o_ref[...] = jnp.reshape(stk / (1.0 + jnp.maximum(axm, bmx)), (1, 1))


_in_specs = (
    [pl.BlockSpec((BM, N), lambda i, s=s: (s * NB + i, 0)) for s in range(S)]
    + [pl.BlockSpec((1, N), lambda i: (0, 0))]
    + [pl.BlockSpec((BM,), lambda i, s=s: (s * NB + i,)) for s in range(S)]
    + [pl.BlockSpec((BM,), lambda i, s=s: (s * NB + i,)) for s in range(S)]
    + [pl.BlockSpec((1, N), lambda i: (0, 0)) for _ in range(4)]
)

_fused = pl.pallas_call(
    _body,
    grid=(NB,),
    in_specs=_in_specs,
    out_specs=pl.BlockSpec((1, 1), lambda i: (0, 0)),
    out_shape=jax.ShapeDtypeStruct((1, 1), jnp.float32),
    scratch_shapes=[pltpu.VMEM((3, 128), jnp.float32)],
)


def kernel(A, b, c, x, Iy, il, iu, l, u):
    del c
    iy = Iy.reshape(M)
    args = ([A] * S + [x.reshape(1, N)] + [b] * S + [iy] * S
            + [l.reshape(1, N), u.reshape(1, N),
               il.reshape(1, N), iu.reshape(1, N)])
    out = _fused(*args)
    return out[0, 0]
